# baseline (speedup 1.0000x reference)
import dataclasses
import jax
import jax.numpy as jnp
from jax import lax
from jax.experimental import pallas as pl
from jax.experimental.pallas import tpu as pltpu
from jax.experimental.pallas import tpu_sc as plsc

N = 4096
NG = 6
NB = NG * NG
ROWW = 40
HD = 128
NC = 2
NS = 16
NW = NC * NS
LN = 16
BM = 512

SPLIT = 2560
NSC = N - SPLIT
IPW = NSC // NW
OCW = 128

BI = 128
BJ = 512
NCHUNK = N // BJ

_cp = pltpu.CompilerParams()
if "needs_layout_passes" in pltpu.CompilerParams.__dataclass_fields__:
    _cp = dataclasses.replace(_cp, needs_layout_passes=False)


def _pool_kernel(obs_i_ref, obs_t_ref, wt_ref, b_ref, out_ref):
    pid = pl.program_id(0)
    xi = obs_i_ref[:, 0:1]
    yi = obs_i_ref[:, 1:2]
    i_glob = pid * BI + jax.lax.broadcasted_iota(jnp.int32, (BI, 1), 0)

    NGRP = (NB + 7) // 8
    accs = [jnp.zeros((BI, BJ), jnp.int32) for _ in range(NGRP)]
    for c in range(NCHUNK):
        xj = obs_t_ref[0:1, c * BJ:(c + 1) * BJ]
        yj = obs_t_ref[1:2, c * BJ:(c + 1) * BJ]
        ox = (xj - xi) + (NG / 2.0)
        oy = (yj - yi) + (NG / 2.0)
        valid = ((jnp.minimum(ox, oy) >= 0.0) &
                 (jnp.maximum(ox, oy) < float(NG)))
        xb = ox.astype(jnp.int32)
        yb = oy.astype(jnp.int32)
        binv = jnp.where(valid, xb * NG + yb, -1)
        w = jnp.left_shift(1, jnp.left_shift(binv & 7, 2))
        g = jnp.right_shift(binv, 3)
        for gi in range(NGRP):
            accs[gi] = accs[gi] + jnp.where(g == gi, w, 0)

    selfhit = jnp.where((xi * 0.0 == 0.0) & (yi * 0.0 == 0.0), 1.0, 0.0)
    occ_cols = []
    for gi in range(NGRP):
        ev = accs[gi] & 0x0F0F0F0F
        od = jnp.right_shift(accs[gi], 4) & 0x0F0F0F0F
        ev = (ev[:, 0:128] + ev[:, 128:256]) + (ev[:, 256:384] + ev[:, 384:512])
        od = (od[:, 0:128] + od[:, 128:256]) + (od[:, 256:384] + od[:, 384:512])
        for r in range(8):
            k = 8 * gi + r
            if k >= NB:
                break
            plane = ev if r % 2 == 0 else od
            byte = jnp.right_shift(plane, 8 * (r // 2)) & 255
            col = jnp.sum(byte.astype(jnp.float32), axis=1, keepdims=True)
            if k == 3 * NG + 3:
                col = col - selfhit
            occ_cols.append(col)
    occ_cols += [jnp.zeros((BI, 1), jnp.float32)] * (ROWW - NB)
    occ = jnp.concatenate(occ_cols, axis=1)
    out_ref[...] = (
        jnp.dot(occ, wt_ref[...], preferred_element_type=jnp.float32)
        + b_ref[...]
    )


def _hist_body(obs_t_hbm, occ_hbm, xs_v, ys_v, occ_v, sem):
    cid = lax.axis_index("core")
    sid = lax.axis_index("subcore")
    wid = cid * NS + sid
    base_i = SPLIT + wid * IPW

    copy_x = pltpu.make_async_copy(obs_t_hbm.at[0], xs_v, sem)
    copy_x.start()
    copy_y = pltpu.make_async_copy(obs_t_hbm.at[1], ys_v, sem)
    copy_y.start()

    zero16 = jnp.zeros((LN,), jnp.float32)

    @pl.loop(0, IPW, step=1)
    def _(r):
        for c in range(OCW // LN):
            occ_v[r, pl.ds(c * LN, LN)] = zero16

    copy_x.wait()
    copy_y.wait()

    lane = lax.broadcasted_iota(jnp.int32, (LN,), 0)
    ones = jnp.ones((LN,), jnp.float32)
    c21 = jnp.full((LN,), 3 * NG + 3, jnp.int32)

    @pl.loop(0, IPW, step=LN)
    def _(ic):
        xi = xs_v[pl.ds(base_i + ic, LN)]
        yi = ys_v[pl.ds(base_i + ic, LN)]
        row = ic + lane

        @pl.loop(0, N, step=LN)
        def _(jc):
            xjv = xs_v[pl.ds(jc, LN)]
            yjv = ys_v[pl.ds(jc, LN)]
            for jj in range(LN):
                ox = (xjv[jj] - xi) + (NG / 2.0)
                oy = (yjv[jj] - yi) + (NG / 2.0)
                m = ((jnp.minimum(ox, oy) >= 0.0) &
                     (jnp.maximum(ox, oy) < float(NG)))
                xb = ox.astype(jnp.int32)
                yb = oy.astype(jnp.int32)
                plsc.addupdate_scatter(occ_v, [row, xb * NG + yb], ones, mask=m)

        selfm = (xi * 0.0 == 0.0) & (yi * 0.0 == 0.0)
        plsc.addupdate_scatter(occ_v, [row, c21], -ones, mask=selfm)

    pltpu.sync_copy(occ_v, occ_hbm.at[pl.ds(wid * IPW, IPW)])


def _occupancy_sc(obs_t):
    mesh = plsc.VectorSubcoreMesh(core_axis_name="core", subcore_axis_name="subcore")
    k = pl.kernel(
        _hist_body,
        out_type=jax.ShapeDtypeStruct((NSC, OCW), jnp.float32),
        mesh=mesh,
        scratch_types=[
            pltpu.VMEM((N,), jnp.float32),
            pltpu.VMEM((N,), jnp.float32),
            pltpu.VMEM((IPW, OCW), jnp.float32),
            pltpu.SemaphoreType.DMA,
        ],
        compiler_params=_cp,
    )
    return k(obs_t)


def _embed_kernel(prev_ref, occ_ref, wt_ref, b_ref, out_ref):
    del prev_ref
    out_ref[...] = (
        jnp.dot(occ_ref[...], wt_ref[...], preferred_element_type=jnp.float32)
        + b_ref[...]
    )


@jax.jit
def kernel(hidden_state, obs1, obs2, W, b):
    del hidden_state, obs1
    obs_t = obs2.T
    wt = jnp.zeros((OCW, HD), jnp.float32).at[:NB].set(W.T)
    b_row = b.reshape(1, HD)

    occ_sc = _occupancy_sc(obs_t)

    out_tc = pl.pallas_call(
        _pool_kernel,
        grid=(SPLIT // BI,),
        in_specs=[
            pl.BlockSpec((BI, 2), lambda i: (i, 0)),
            pl.BlockSpec((2, N), lambda i: (0, 0)),
            pl.BlockSpec((ROWW, HD), lambda i: (0, 0)),
            pl.BlockSpec((1, HD), lambda i: (0, 0)),
        ],
        out_specs=pl.BlockSpec((BI, HD), lambda i: (i, 0)),
        out_shape=jax.ShapeDtypeStruct((N, HD), jnp.float32),
    )(obs2[:SPLIT], obs_t, wt, b_row)

    out = pl.pallas_call(
        _embed_kernel,
        grid=(NSC // BM,),
        in_specs=[
            pl.BlockSpec((BM, HD), lambda i: (i + SPLIT // BM, 0)),
            pl.BlockSpec((BM, OCW), lambda i: (i, 0)),
            pl.BlockSpec((OCW, HD), lambda i: (0, 0)),
            pl.BlockSpec((1, HD), lambda i: (0, 0)),
        ],
        out_specs=pl.BlockSpec((BM, HD), lambda i: (i + SPLIT // BM, 0)),
        out_shape=jax.ShapeDtypeStruct((N, HD), jnp.float32),
        input_output_aliases={0: 0},
    )(out_tc, occ_sc, wt, b_row)

    return out


# speedup vs baseline: 1.0723x; 1.0723x over previous
import dataclasses
import jax
import jax.numpy as jnp
from jax import lax
from jax.experimental import pallas as pl
from jax.experimental.pallas import tpu as pltpu
from jax.experimental.pallas import tpu_sc as plsc

N = 4096
NG = 6
NB = NG * NG
ROWW = 40
HD = 128
NC = 2
NS = 16
NW = NC * NS
LN = 16
BM = 512

SPLIT = 2560
NSC = N - SPLIT
IPW = NSC // NW
OCW = 128

BI = 128
BJ = 512
NCHUNK = N // BJ

_cp = pltpu.CompilerParams()
if "needs_layout_passes" in pltpu.CompilerParams.__dataclass_fields__:
    _cp = dataclasses.replace(_cp, needs_layout_passes=False)


def _pool_kernel(obs_i_ref, obs_t_ref, wt_ref, b_ref, out_ref):
    pid = pl.program_id(0)
    xi = obs_i_ref[:, 0:1]
    yi = obs_i_ref[:, 1:2]
    i_glob = pid * BI + jax.lax.broadcasted_iota(jnp.int32, (BI, 1), 0)

    NGRP = (NB + 7) // 8
    accs = [jnp.zeros((BI, BJ), jnp.int32) for _ in range(NGRP)]
    for c in range(NCHUNK):
        xj = obs_t_ref[0:1, c * BJ:(c + 1) * BJ]
        yj = obs_t_ref[1:2, c * BJ:(c + 1) * BJ]
        ox = (xj - xi) + (NG / 2.0)
        oy = (yj - yi) + (NG / 2.0)
        valid = ((jnp.minimum(ox, oy) >= 0.0) &
                 (jnp.maximum(ox, oy) < float(NG)))
        xb = ox.astype(jnp.int32)
        yb = oy.astype(jnp.int32)
        binv = jnp.where(valid, xb * NG + yb, -1)
        w = jnp.left_shift(1, jnp.left_shift(binv & 7, 2))
        g = jnp.right_shift(binv, 3)
        for gi in range(NGRP):
            accs[gi] = accs[gi] + jnp.where(g == gi, w, 0)

    selfhit = jnp.where((xi * 0.0 == 0.0) & (yi * 0.0 == 0.0), 1.0, 0.0)
    occ_cols = []
    for gi in range(NGRP):
        ev = accs[gi] & 0x0F0F0F0F
        od = jnp.right_shift(accs[gi], 4) & 0x0F0F0F0F
        ev = (ev[:, 0:128] + ev[:, 128:256]) + (ev[:, 256:384] + ev[:, 384:512])
        od = (od[:, 0:128] + od[:, 128:256]) + (od[:, 256:384] + od[:, 384:512])
        for r in range(8):
            k = 8 * gi + r
            if k >= NB:
                break
            plane = ev if r % 2 == 0 else od
            byte = jnp.right_shift(plane, 8 * (r // 2)) & 255
            col = jnp.sum(byte.astype(jnp.float32), axis=1, keepdims=True)
            if k == 3 * NG + 3:
                col = col - selfhit
            occ_cols.append(col)
    occ_cols += [jnp.zeros((BI, 1), jnp.float32)] * (ROWW - NB)
    occ = jnp.concatenate(occ_cols, axis=1)
    out_ref[...] = (
        jnp.dot(occ, wt_ref[...], preferred_element_type=jnp.float32)
        + b_ref[...]
    )


def _hist_body(obs_t_hbm, occ_hbm, xs_v, ys_v, occ_v, sem):
    cid = lax.axis_index("core")
    sid = lax.axis_index("subcore")
    wid = cid * NS + sid
    base_i = SPLIT + wid * IPW

    copy_x = pltpu.make_async_copy(obs_t_hbm.at[0], xs_v, sem)
    copy_x.start()
    copy_y = pltpu.make_async_copy(obs_t_hbm.at[1], ys_v, sem)
    copy_y.start()

    zero16 = jnp.zeros((LN,), jnp.float32)

    @pl.loop(0, IPW * OCW, step=LN)
    def _(k):
        occ_v[pl.ds(k, LN)] = zero16

    copy_x.wait()
    copy_y.wait()

    lane = lax.broadcasted_iota(jnp.int32, (LN,), 0)
    ones = jnp.ones((LN,), jnp.float32)

    @pl.loop(0, IPW, step=LN)
    def _(ic):
        xi = xs_v[pl.ds(base_i + ic, LN)]
        yi = ys_v[pl.ds(base_i + ic, LN)]
        rowbase = (ic + lane) * OCW

        @pl.loop(0, N, step=LN)
        def _(jc):
            xjv = xs_v[pl.ds(jc, LN)]
            yjv = ys_v[pl.ds(jc, LN)]
            for jj in range(LN):
                ox = (xjv[jj] - xi) + (NG / 2.0)
                oy = (yjv[jj] - yi) + (NG / 2.0)
                m = ((jnp.minimum(ox, oy) >= 0.0) &
                     (jnp.maximum(ox, oy) < float(NG)))
                xb = ox.astype(jnp.int32)
                yb = oy.astype(jnp.int32)
                idx = rowbase + (xb * NG + yb)
                plsc.addupdate_scatter(occ_v, [idx], ones, mask=m)

        selfm = (xi * 0.0 == 0.0) & (yi * 0.0 == 0.0)
        plsc.addupdate_scatter(occ_v, [rowbase + (3 * NG + 3)], -ones, mask=selfm)

    pltpu.sync_copy(occ_v, occ_hbm.at[pl.ds(wid * IPW * OCW, IPW * OCW)])


def _occupancy_sc(obs_t):
    mesh = plsc.VectorSubcoreMesh(core_axis_name="core", subcore_axis_name="subcore")
    k = pl.kernel(
        _hist_body,
        out_type=jax.ShapeDtypeStruct((NSC * OCW,), jnp.float32),
        mesh=mesh,
        scratch_types=[
            pltpu.VMEM((N,), jnp.float32),
            pltpu.VMEM((N,), jnp.float32),
            pltpu.VMEM((IPW * OCW,), jnp.float32),
            pltpu.SemaphoreType.DMA,
        ],
        compiler_params=_cp,
    )
    return k(obs_t)


def _embed_kernel(prev_ref, occ_ref, wt_ref, b_ref, out_ref):
    del prev_ref
    out_ref[...] = (
        jnp.dot(occ_ref[...], wt_ref[...], preferred_element_type=jnp.float32)
        + b_ref[...]
    )


@jax.jit
def kernel(hidden_state, obs1, obs2, W, b):
    del hidden_state, obs1
    obs_t = obs2.T
    wt = jnp.zeros((OCW, HD), jnp.float32).at[:NB].set(W.T)
    b_row = b.reshape(1, HD)

    occ_sc = _occupancy_sc(obs_t).reshape(NSC, OCW)

    out_tc = pl.pallas_call(
        _pool_kernel,
        grid=(SPLIT // BI,),
        in_specs=[
            pl.BlockSpec((BI, 2), lambda i: (i, 0)),
            pl.BlockSpec((2, N), lambda i: (0, 0)),
            pl.BlockSpec((ROWW, HD), lambda i: (0, 0)),
            pl.BlockSpec((1, HD), lambda i: (0, 0)),
        ],
        out_specs=pl.BlockSpec((BI, HD), lambda i: (i, 0)),
        out_shape=jax.ShapeDtypeStruct((N, HD), jnp.float32),
    )(obs2[:SPLIT], obs_t, wt, b_row)

    out = pl.pallas_call(
        _embed_kernel,
        grid=(NSC // BM,),
        in_specs=[
            pl.BlockSpec((BM, HD), lambda i: (i + SPLIT // BM, 0)),
            pl.BlockSpec((BM, OCW), lambda i: (i, 0)),
            pl.BlockSpec((OCW, HD), lambda i: (0, 0)),
            pl.BlockSpec((1, HD), lambda i: (0, 0)),
        ],
        out_specs=pl.BlockSpec((BM, HD), lambda i: (i + SPLIT // BM, 0)),
        out_shape=jax.ShapeDtypeStruct((N, HD), jnp.float32),
        input_output_aliases={0: 0},
    )(out_tc, occ_sc, wt, b_row)

    return out
